# baseline (speedup 1.0000x reference)
import jax
import jax.numpy as jnp
from jax.experimental import pallas as pl
from jax.experimental.pallas import tpu as pltpu

_BN = 1280


def _fused_kernel(tok_ref, emb_ref, W1_ref, b1_ref, W2_ref, b2_ref,
                  ntpW_ref, ntpb_ref, out_ref, v_ref, sem, *, nexp):
    i = pl.program_id(0)

    @pl.when(i == 0)
    def _gather():
        tok = tok_ref[0, tok_ref.shape[1] - 1]
        cp = pltpu.make_async_copy(
            emb_ref.at[pl.ds(tok, 1), :], v_ref, sem)
        cp.start()
        cp.wait()

    @pl.when(i < nexp)
    def _expert():
        t = jnp.maximum(
            jnp.dot(v_ref[...], W1_ref[0],
                    preferred_element_type=jnp.float32) + b1_ref[0], 0.0)
        v_ref[...] = (
            jnp.dot(t, W2_ref[0], preferred_element_type=jnp.float32)
            + b2_ref[0])

    @pl.when(i >= nexp)
    def _project():
        out_ref[...] = (
            jnp.dot(v_ref[...], ntpW_ref[...],
                    preferred_element_type=jnp.float32) + ntpb_ref[...])


def kernel(X, emb, W1, b1, W2, b2, ntp_W, ntp_b):
    vocab, d = emb.shape
    nblocks, _, dff = W1.shape
    nexp = nblocks - 1
    nv = vocab // _BN

    tok = X.astype(jnp.int32)
    b1r = b1.reshape(nblocks, 1, dff)
    b2r = b2.reshape(nblocks, 1, d)

    def _eidx(i):
        return jnp.minimum(i, nexp - 1) + 1

    def _vidx(i):
        return jnp.maximum(i - nexp, 0)

    import functools
    body = functools.partial(_fused_kernel, nexp=nexp)
    logits = pl.pallas_call(
        body,
        grid=(nexp + nv,),
        in_specs=[
            pl.BlockSpec(memory_space=pltpu.SMEM),
            pl.BlockSpec(memory_space=pl.ANY),
            pl.BlockSpec((1, d, dff), lambda i: (_eidx(i), 0, 0)),
            pl.BlockSpec((1, 1, dff), lambda i: (_eidx(i), 0, 0)),
            pl.BlockSpec((1, dff, d), lambda i: (_eidx(i), 0, 0)),
            pl.BlockSpec((1, 1, d), lambda i: (_eidx(i), 0, 0)),
            pl.BlockSpec((d, _BN), lambda i: (0, _vidx(i))),
            pl.BlockSpec((1, _BN), lambda i: (0, _vidx(i))),
        ],
        out_specs=pl.BlockSpec((1, _BN), lambda i: (0, _vidx(i))),
        out_shape=jax.ShapeDtypeStruct((1, vocab), jnp.float32),
        scratch_shapes=[pltpu.VMEM((1, d), jnp.float32),
                        pltpu.SemaphoreType.DMA],
    )(tok, emb, W1, b1r, W2, b2r, ntp_W, ntp_b.reshape(1, vocab))
    return logits


# speedup vs baseline: 1.0475x; 1.0475x over previous
import jax
import jax.numpy as jnp
from jax.experimental import pallas as pl
from jax.experimental.pallas import tpu as pltpu

_BN = 3200


def _chain_kernel(tok_ref, emb_ref, W1_ref, b1_ref, W2_ref, b2_ref,
                  out_ref, v_ref, sem):
    e = pl.program_id(0)
    ne = pl.num_programs(0)

    @pl.when(e == 0)
    def _gather():
        tok = tok_ref[0, tok_ref.shape[1] - 1]
        cp = pltpu.make_async_copy(
            emb_ref.at[pl.ds(tok, 1), :], v_ref, sem)
        cp.start()
        cp.wait()

    b1 = b1_ref[pl.ds(e + 1, 1), :]
    b2 = b2_ref[pl.ds(e + 1, 1), :]
    t = jnp.maximum(
        jnp.dot(v_ref[...], W1_ref[0], preferred_element_type=jnp.float32)
        + b1, 0.0)
    v_ref[...] = (
        jnp.dot(t, W2_ref[0], preferred_element_type=jnp.float32) + b2)

    @pl.when(e == ne - 1)
    def _emit():
        out_ref[...] = v_ref[...]


def _ntp_kernel(v_ref, W_ref, b_ref, out_ref):
    out_ref[...] = (
        jnp.dot(v_ref[...], W_ref[...], preferred_element_type=jnp.float32)
        + b_ref[...])


def kernel(X, emb, W1, b1, W2, b2, ntp_W, ntp_b):
    vocab, d = emb.shape
    nblocks, _, dff = W1.shape
    nexp = nblocks - 1

    tok = X.astype(jnp.int32)

    v = pl.pallas_call(
        _chain_kernel,
        grid=(nexp,),
        in_specs=[
            pl.BlockSpec(memory_space=pltpu.SMEM),
            pl.BlockSpec(memory_space=pl.ANY),
            pl.BlockSpec((1, d, dff), lambda e: (e + 1, 0, 0)),
            pl.BlockSpec((nblocks, dff), lambda e: (0, 0)),
            pl.BlockSpec((1, dff, d), lambda e: (e + 1, 0, 0)),
            pl.BlockSpec((nblocks, d), lambda e: (0, 0)),
        ],
        out_specs=pl.BlockSpec((1, d), lambda e: (0, 0)),
        out_shape=jax.ShapeDtypeStruct((1, d), jnp.float32),
        scratch_shapes=[pltpu.VMEM((1, d), jnp.float32),
                        pltpu.SemaphoreType.DMA],
    )(tok, emb, W1, b1, W2, b2)

    nv = vocab // _BN
    logits = pl.pallas_call(
        _ntp_kernel,
        grid=(nv,),
        in_specs=[
            pl.BlockSpec((1, d), lambda j: (0, 0)),
            pl.BlockSpec((d, _BN), lambda j: (0, j)),
            pl.BlockSpec((1, _BN), lambda j: (0, j)),
        ],
        out_specs=pl.BlockSpec((1, _BN), lambda j: (0, j)),
        out_shape=jax.ShapeDtypeStruct((1, vocab), jnp.float32),
    )(v, ntp_W, ntp_b.reshape(1, vocab))
    return logits


# speedup vs baseline: 1.0603x; 1.0122x over previous
import functools

import jax
import jax.numpy as jnp
from jax.experimental import pallas as pl
from jax.experimental.pallas import tpu as pltpu

_BN = 3200


def _fused_kernel(tok_ref, emb_ref, W1_ref, b1_ref, W2_ref, b2_ref,
                  ntpW_ref, ntpb_ref, out_ref, v_ref, acc_ref, sem,
                  *, nexp, bf):
    i = pl.program_id(0)
    nchain = 2 * nexp

    @pl.when(i == 0)
    def _gather():
        tok = tok_ref[0, tok_ref.shape[1] - 1]
        cp = pltpu.make_async_copy(
            emb_ref.at[pl.ds(tok, 1), :], v_ref, sem)
        cp.start()
        cp.wait()

    @pl.when(i < nchain)
    def _expert_half():
        e = i // 2
        h = i % 2
        b1h = b1_ref[pl.ds(i + 2, 1), :]
        t = jnp.maximum(
            jnp.dot(v_ref[...], W1_ref[0],
                    preferred_element_type=jnp.float32) + b1h, 0.0)
        part = jnp.dot(t, W2_ref[0], preferred_element_type=jnp.float32)

        @pl.when(h == 0)
        def _():
            acc_ref[...] = part

        @pl.when(h == 1)
        def _():
            v_ref[...] = acc_ref[...] + part + b2_ref[pl.ds(e + 1, 1), :]

    @pl.when(i >= nchain)
    def _project():
        out_ref[...] = (
            jnp.dot(v_ref[...], ntpW_ref[...],
                    preferred_element_type=jnp.float32) + ntpb_ref[...])


def kernel(X, emb, W1, b1, W2, b2, ntp_W, ntp_b):
    vocab, d = emb.shape
    nblocks, _, dff = W1.shape
    nexp = nblocks - 1
    bf = dff // 2
    nchain = 2 * nexp
    nv = vocab // _BN

    tok = X.astype(jnp.int32)
    b1r = b1.reshape(nblocks * 2, bf)

    def _e(i):
        return jnp.minimum(i // 2, nexp - 1) + 1

    def _h(i):
        return jnp.minimum(i, nchain - 1) % 2

    def _j(i):
        return jnp.maximum(i - nchain, 0)

    body = functools.partial(_fused_kernel, nexp=nexp, bf=bf)
    logits = pl.pallas_call(
        body,
        grid=(nchain + nv,),
        in_specs=[
            pl.BlockSpec(memory_space=pltpu.SMEM),
            pl.BlockSpec(memory_space=pl.ANY),
            pl.BlockSpec((1, d, bf), lambda i: (_e(i), 0, _h(i))),
            pl.BlockSpec((nblocks * 2, bf), lambda i: (0, 0)),
            pl.BlockSpec((1, bf, d), lambda i: (_e(i), _h(i), 0)),
            pl.BlockSpec((nblocks, d), lambda i: (0, 0)),
            pl.BlockSpec((d, _BN), lambda i: (0, _j(i))),
            pl.BlockSpec((1, _BN), lambda i: (0, _j(i))),
        ],
        out_specs=pl.BlockSpec((1, _BN), lambda i: (0, _j(i))),
        out_shape=jax.ShapeDtypeStruct((1, vocab), jnp.float32),
        scratch_shapes=[pltpu.VMEM((1, d), jnp.float32),
                        pltpu.VMEM((1, d), jnp.float32),
                        pltpu.SemaphoreType.DMA],
    )(tok, emb, W1, b1r, W2, b2, ntp_W, ntp_b.reshape(1, vocab))
    return logits
